# baseline (speedup 1.0000x reference)
import functools

import jax
import jax.numpy as jnp
from jax.experimental import pallas as pl
from jax.experimental.pallas import tpu as pltpu

_HUFFMAN_PATHS = ((0, 1), (1, 0), (0, 0, 1), (1, 1, 0))
_NCOL = 16
_SPLIT = 4


def _coeff_tables(n):
    r = jax.lax.broadcasted_iota(jnp.int32, (8, _NCOL), 0)
    c = jax.lax.broadcasted_iota(jnp.int32, (8, _NCOL), 1)
    a = jnp.zeros((8, _NCOL), jnp.float32)
    b = jnp.zeros((8, _NCOL), jnp.float32)
    off = 0
    for w, path in enumerate(_HUFFMAN_PATHS):
        lw = len(path)
        coeff = 1.0 / (lw * n)
        a = jnp.where((r == w) & (c >= off) & (c < off + lw), coeff, a)
        ones = [off + j for j, bit in enumerate(path) if bit == 1]
        b = jnp.where(
            (r == w) & (c >= ones[0]) & (c < ones[-1] + 1), coeff, b
        )
        off += lw
    return a, b


def _body(*refs):
    h_refs = refs[:_SPLIT]
    tw_refs = refs[_SPLIT : 2 * _SPLIT]
    w0_ref, w1_ref, w2_ref, w3_ref, out_ref, wt_ref, wstk_ref = refs[2 * _SPLIT :]
    bm = h_refs[0].shape[0]
    n = pl.num_programs(0) * bm * _SPLIT

    @pl.when(pl.program_id(0) == 0)
    def _():
        wstk_ref[...] = jnp.zeros_like(wstk_ref)
        wstk_ref[0:2, :] = w0_ref[...]
        wstk_ref[2:4, :] = w1_ref[...]
        wstk_ref[4:7, :] = w2_ref[...]
        wstk_ref[7:10, :] = w3_ref[...]
        eye = (
            jax.lax.broadcasted_iota(jnp.int32, (_NCOL, _NCOL), 0)
            == jax.lax.broadcasted_iota(jnp.int32, (_NCOL, _NCOL), 1)
        ).astype(jnp.float32)
        wt_ref[...] = jax.lax.dot_general(
            wstk_ref[...], eye, (((0,), (0,)), ((), ())),
            preferred_element_type=jnp.float32,
        )
        out_ref[0, 0] = 0.0

    a_tab, b_tab = _coeff_tables(n)
    wt = wt_ref[...]
    acc = jnp.zeros((8, _NCOL), jnp.float32)
    for h_ref, tw_ref in zip(h_refs, tw_refs):
        h = h_ref[...]
        tw = tw_ref[...]
        x = jnp.dot(h, wt, preferred_element_type=jnp.float32)
        soft = jnp.maximum(x, 0.0) + jnp.log1p(jnp.exp(-jnp.abs(x)))
        onehot = (
            tw == jax.lax.broadcasted_iota(jnp.int32, (bm, 8), 1)
        ).astype(jnp.float32)
        s_tab = jax.lax.dot_general(
            onehot, soft, (((0,), (0,)), ((), ())),
            preferred_element_type=jnp.float32,
        )
        x_tab = jax.lax.dot_general(
            onehot, x, (((0,), (0,)), ((), ())),
            preferred_element_type=jnp.float32,
        )
        acc = acc + a_tab * s_tab - b_tab * x_tab
    out_ref[0, 0] += jnp.sum(acc)


@functools.partial(jax.jit, static_argnames=("interpret", "bm"))
def kernel(hidden, target_words, W_0, W_1, W_2, W_3, interpret=False, bm=512):
    batch, hdim = hidden.shape
    slab_blocks = batch // _SPLIT // bm
    tw2d = target_words.astype(jnp.int32).reshape(batch, 1)

    h_specs = [
        pl.BlockSpec((bm, hdim), lambda i, s=s: (s * slab_blocks + i, 0))
        for s in range(_SPLIT)
    ]
    tw_specs = [
        pl.BlockSpec((bm, 1), lambda i, s=s: (s * slab_blocks + i, 0))
        for s in range(_SPLIT)
    ]
    full = lambda shape: pl.BlockSpec(shape, lambda i: (0, 0))
    out = pl.pallas_call(
        _body,
        grid=(slab_blocks,),
        in_specs=h_specs
        + tw_specs
        + [full(W_0.shape), full(W_1.shape), full(W_2.shape), full(W_3.shape)],
        out_specs=pl.BlockSpec(
            (1, 1), lambda i: (0, 0), memory_space=pltpu.SMEM
        ),
        out_shape=jax.ShapeDtypeStruct((1, 1), jnp.float32),
        scratch_shapes=[
            pltpu.VMEM((hdim, _NCOL), jnp.float32),
            pltpu.VMEM((_NCOL, hdim), jnp.float32),
        ],
        interpret=interpret,
    )(*([hidden] * _SPLIT + [tw2d] * _SPLIT + [W_0, W_1, W_2, W_3]))
    return out[0, 0]
